# Initial kernel scaffold; baseline (speedup 1.0000x reference)
import jax
import jax.numpy as jnp
from jax.experimental import pallas as pl


def kernel(input_ids, emb, W, b):
    raise NotImplementedError("write your pallas kernel here")


# speedup vs baseline: 1.3137x; 1.3137x over previous
import functools

import jax
import jax.numpy as jnp
from jax import lax
from jax.experimental import pallas as pl
from jax.experimental.pallas import tpu as pltpu
from jax.experimental.pallas import tpu_sc as plsc

V = 16
D = 16
NC = 2
NS = 16
NW = NC * NS
G = 128
K = 16
CHUNK = K * G


def _table_body(emb_ref, w_ref, b_ref, out_ref):
    out_ref[...] = lax.dot_general(
        emb_ref[...], w_ref[...],
        dimension_numbers=(((1,), (1,)), ((), ())),
        preferred_element_type=jnp.float32,
    ) + b_ref[...]


def _build_table(emb, W, b):
    b2 = jnp.broadcast_to(b[None, :], (V, V))
    return pl.pallas_call(
        _table_body,
        out_shape=jax.ShapeDtypeStruct((V, V), jnp.float32),
    )(emb, W, b2)


@functools.lru_cache(maxsize=None)
def _make_sc_gather(n_rows: int):
    assert n_rows % (NW * CHUNK) == 0
    per_w = n_rows // NW
    steps = per_w // CHUNK
    mesh = plsc.VectorSubcoreMesh(core_axis_name="c", subcore_axis_name="s")

    @functools.partial(
        pl.kernel,
        out_type=jax.ShapeDtypeStruct((n_rows, D), jnp.float32),
        mesh=mesh,
        compiler_params=pltpu.CompilerParams(use_tc_tiling_on_sc=False),
        scratch_types=[
            pltpu.VMEM((K, G), jnp.int32),
            pltpu.VMEM((CHUNK, D), jnp.float32),
            pltpu.SemaphoreType.DMA,
        ],
    )
    def sc_gather(table_hbm, idx_hbm, out_hbm, idx_v, rows_v, sem):
        wid = lax.axis_index("s") * NC + lax.axis_index("c")

        @pl.loop(0, steps)
        def _(step):
            base = pl.multiple_of(wid * per_w + step * CHUNK, CHUNK)
            pltpu.sync_copy(idx_hbm.at[pl.ds(pl.multiple_of(base // G, K), K)], idx_v)
            copies = [
                pltpu.async_copy(
                    table_hbm.at[idx_v.at[j]],
                    rows_v.at[pl.ds(j * G, G)],
                    sem,
                )
                for j in range(K)
            ]
            for c in copies:
                c.wait()
            pltpu.sync_copy(rows_v, out_hbm.at[pl.ds(base, CHUNK)])

    return sc_gather


def kernel(input_ids, emb, W, b):
    batch, seq = input_ids.shape
    n = batch * seq
    ids2 = input_ids.reshape(n // G, G).astype(jnp.int32)
    table = _build_table(emb, W, b)
    out = _make_sc_gather(n)(table, ids2)
    return out.reshape(batch, seq, V)


# speedup vs baseline: 4.6931x; 3.5725x over previous
import dataclasses
import functools

import jax
import jax.numpy as jnp
from jax import lax
from jax.experimental import pallas as pl
from jax.experimental.pallas import tpu as pltpu
from jax.experimental.pallas import tpu_sc as plsc

V = 16
D = 16
NC = 2
NS = 16
NW = NC * NS
CHUNK = 3200


def _table_body(emb_ref, w_ref, b_ref, out_ref):
    out_ref[...] = lax.dot_general(
        emb_ref[...], w_ref[...],
        dimension_numbers=(((1,), (1,)), ((), ())),
        preferred_element_type=jnp.float32,
    ) + b_ref[...]


def _build_table(emb, W, b):
    b2 = jnp.broadcast_to(b[None, :], (V, V))
    return pl.pallas_call(
        _table_body,
        out_shape=jax.ShapeDtypeStruct((V, V), jnp.float32),
    )(emb, W, b2)


def _sc_compiler_params():
    cp = pltpu.CompilerParams(use_tc_tiling_on_sc=False)
    if "needs_layout_passes" in pltpu.CompilerParams.__dataclass_fields__:
        cp = dataclasses.replace(cp, needs_layout_passes=False)
    return cp


@functools.lru_cache(maxsize=None)
def _make_sc_gather(n_rows: int):
    assert n_rows % (NW * CHUNK) == 0
    per_w = n_rows // NW
    steps = per_w // CHUNK
    assert steps % 2 == 0
    groups = CHUNK // 16
    mesh = plsc.VectorSubcoreMesh(core_axis_name="c", subcore_axis_name="s")

    @functools.partial(
        pl.kernel,
        out_type=jax.ShapeDtypeStruct((n_rows * D,), jnp.float32),
        mesh=mesh,
        compiler_params=_sc_compiler_params(),
        scratch_types=[
            pltpu.VMEM((V * D,), jnp.float32),
            pltpu.VMEM((2, CHUNK), jnp.int32),
            pltpu.VMEM((2, CHUNK * D), jnp.float32),
            pltpu.SemaphoreType.DMA,
            pltpu.SemaphoreType.DMA,
        ],
    )
    def sc_gather(table_hbm, idx_hbm, out_hbm, table_v, idx_v, out_v, sem_in, sem_out):
        wid = lax.axis_index("s") * NC + lax.axis_index("c")
        row0 = pl.multiple_of(wid * per_w, CHUNK)
        out_iota = lax.iota(jnp.int32, 16) * D

        pltpu.sync_copy(table_hbm, table_v)
        for b in range(2):
            pltpu.async_copy(
                idx_hbm.at[pl.ds(pl.multiple_of(row0 + b * CHUNK, CHUNK), CHUNK)],
                idx_v.at[b], sem_in)

        @pl.loop(0, steps, step=2)
        def _(s0):
            for b in range(2):
                s = s0 + b
                pltpu.make_async_copy(
                    idx_hbm.at[pl.ds(0, CHUNK)], idx_v.at[b], sem_in).wait()
                @pl.when(s0 >= 2)
                def _():
                    pltpu.make_async_copy(
                        out_v.at[b], out_hbm.at[pl.ds(0, CHUNK * D)], sem_out).wait()

                @pl.loop(0, groups)
                def _(g):
                    ids = idx_v[b, pl.ds(g * 16, 16)]
                    in_base = ids * D
                    out_base = out_iota + g * (16 * D)
                    for c in range(D):
                        vals = plsc.load_gather(table_v, [in_base + c])
                        plsc.store_scatter(out_v.at[b], [out_base + c], vals)

                pltpu.async_copy(
                    out_v.at[b],
                    out_hbm.at[pl.ds(pl.multiple_of((row0 + s * CHUNK) * D, CHUNK * D),
                                     CHUNK * D)],
                    sem_out)

                @pl.when(s + 2 < steps)
                def _():
                    pltpu.async_copy(
                        idx_hbm.at[pl.ds(pl.multiple_of(row0, CHUNK) + (s + 2) * CHUNK,
                                         CHUNK)],
                        idx_v.at[b], sem_in)

        for b in range(2):
            pltpu.make_async_copy(
                out_v.at[b], out_hbm.at[pl.ds(0, CHUNK * D)], sem_out).wait()

    return sc_gather


def kernel(input_ids, emb, W, b):
    batch, seq = input_ids.shape
    n = batch * seq
    ids = input_ids.reshape(n).astype(jnp.int32)
    table = _build_table(emb, W, b).reshape(V * D)
    out = _make_sc_gather(n)(table, ids)
    return out.reshape(batch, seq, V)


# speedup vs baseline: 5.7813x; 1.2319x over previous
import dataclasses
import functools

import jax
import jax.numpy as jnp
from jax import lax
from jax.experimental import pallas as pl
from jax.experimental.pallas import tpu as pltpu
from jax.experimental.pallas import tpu_sc as plsc

V = 16
D = 16
NC = 2
NS = 16
NW = NC * NS
CHUNK = 3200


def _table_body(emb_ref, w_ref, b_ref, out_ref):
    out_ref[...] = lax.dot_general(
        emb_ref[...], w_ref[...],
        dimension_numbers=(((1,), (1,)), ((), ())),
        preferred_element_type=jnp.float32,
    ) + b_ref[...]


def _build_table(emb, W, b):
    b2 = jnp.broadcast_to(b[None, :], (V, V))
    return pl.pallas_call(
        _table_body,
        out_shape=jax.ShapeDtypeStruct((V, V), jnp.float32),
    )(emb, W, b2)


def _sc_compiler_params():
    cp = pltpu.CompilerParams(use_tc_tiling_on_sc=False)
    if "needs_layout_passes" in pltpu.CompilerParams.__dataclass_fields__:
        cp = dataclasses.replace(cp, needs_layout_passes=False)
    return cp


@functools.lru_cache(maxsize=None)
def _make_sc_gather(n_rows: int):
    assert n_rows % (NW * CHUNK) == 0
    per_w = n_rows // NW
    steps = per_w // CHUNK
    assert steps % 2 == 0
    groups = CHUNK // 16
    mesh = plsc.VectorSubcoreMesh(core_axis_name="c", subcore_axis_name="s")

    @functools.partial(
        pl.kernel,
        out_type=jax.ShapeDtypeStruct((n_rows * D,), jnp.float32),
        mesh=mesh,
        compiler_params=_sc_compiler_params(),
        scratch_types=[
            pltpu.VMEM((V * D,), jnp.float32),
            pltpu.VMEM((2, CHUNK), jnp.int32),
            pltpu.VMEM((2, CHUNK * D), jnp.float32),
            pltpu.SemaphoreType.DMA,
            pltpu.SemaphoreType.DMA,
        ],
    )
    def sc_gather(table_hbm, idx_hbm, out_hbm, table_v, idx_v, out_v, sem_in, sem_out):
        wid = lax.axis_index("s") * NC + lax.axis_index("c")
        row0 = pl.multiple_of(wid * per_w, CHUNK)
        out_iota = lax.iota(jnp.int32, 16) * D

        pltpu.sync_copy(table_hbm, table_v)
        for b in range(2):
            pltpu.async_copy(
                idx_hbm.at[pl.ds(pl.multiple_of(row0 + b * CHUNK, CHUNK), CHUNK)],
                idx_v.at[b], sem_in)

        @pl.loop(0, steps, step=2)
        def _(s0):
            for b in range(2):
                s = s0 + b
                pltpu.make_async_copy(
                    idx_hbm.at[pl.ds(0, CHUNK)], idx_v.at[b], sem_in).wait()
                @pl.when(s0 >= 2)
                def _():
                    pltpu.make_async_copy(
                        out_v.at[b], out_hbm.at[pl.ds(0, CHUNK * D)], sem_out).wait()

                @plsc.parallel_loop(0, groups, unroll=4)
                def _(g):
                    ids = idx_v[b, pl.ds(g * 16, 16)]
                    in_base = ids * D
                    out_base = out_iota + g * (16 * D)
                    for c in range(D):
                        vals = plsc.load_gather(table_v, [in_base + c])
                        plsc.store_scatter(out_v.at[b], [out_base + c], vals)

                pltpu.async_copy(
                    out_v.at[b],
                    out_hbm.at[pl.ds(pl.multiple_of((row0 + s * CHUNK) * D, CHUNK * D),
                                     CHUNK * D)],
                    sem_out)

                @pl.when(s + 2 < steps)
                def _():
                    pltpu.async_copy(
                        idx_hbm.at[pl.ds(pl.multiple_of(row0, CHUNK) + (s + 2) * CHUNK,
                                         CHUNK)],
                        idx_v.at[b], sem_in)

        for b in range(2):
            pltpu.make_async_copy(
                out_v.at[b], out_hbm.at[pl.ds(0, CHUNK * D)], sem_out).wait()

    return sc_gather


def kernel(input_ids, emb, W, b):
    batch, seq = input_ids.shape
    n = batch * seq
    ids = input_ids.reshape(n).astype(jnp.int32)
    table = _build_table(emb, W, b).reshape(V * D)
    out = _make_sc_gather(n)(table, ids)
    return out.reshape(batch, seq, V)
